# baseline (speedup 1.0000x reference)
import jax
import jax.numpy as jnp
from jax import lax
from jax.experimental import pallas as pl
from jax.experimental.pallas import tpu as pltpu
from jax.experimental.pallas import tpu_sc as plsc

N = 100000
NW = 32
CHUNK = 3136
LAST = N - (NW - 1) * CHUNK
NZ = 119
TAB = 128
L = 16


def _rsqrt16(s):
    y = jnp.where(s < jnp.float32(0.45), jnp.float32(1.8),
                  jnp.where(s < jnp.float32(1.8), jnp.float32(1.0),
                            jnp.float32(0.45)))
    for _ in range(6):
        y = y * (jnp.float32(1.5) - jnp.float32(0.5) * s * y * y)
    return y


def _scale_body(x_hbm, z_hbm, tab_hbm, out_hbm,
                x_v, z_v, tab_v, rs_v, o_v,
                sem_z0, sem_z1, sem_x0, sem_x1, sem_o0, sem_o1, sem_t):
    cid = lax.axis_index("c")
    sid = lax.axis_index("s")
    wid = sid * 2 + cid
    base = wid * CHUNK

    def run(n):
        h = n // 2
        cz0 = pltpu.make_async_copy(z_hbm.at[pl.ds(base, h)],
                                    z_v.at[pl.ds(0, h)], sem_z0)
        cx0 = pltpu.make_async_copy(x_hbm.at[pl.ds(base, h)],
                                    x_v.at[pl.ds(0, h)], sem_x0)
        cz1 = pltpu.make_async_copy(z_hbm.at[pl.ds(base + h, n - h)],
                                    z_v.at[pl.ds(h, n - h)], sem_z1)
        cx1 = pltpu.make_async_copy(x_hbm.at[pl.ds(base + h, n - h)],
                                    x_v.at[pl.ds(h, n - h)], sem_x1)
        ct = pltpu.make_async_copy(tab_hbm, tab_v.at[pl.ds(0, NZ)], sem_t)
        cz0.start()
        cx0.start()
        ct.start()
        cz1.start()
        cx1.start()

        ct.wait()
        for j in range(TAB // L):
            rs_v[pl.ds(j * L, L)] = _rsqrt16(tab_v[pl.ds(j * L, L)])

        def gather_scale(lo, hi):
            @plsc.parallel_loop(lo, hi, step=L, unroll=16)
            def _body(i):
                idx = z_v[pl.ds(i, L)]
                g = plsc.load_gather(rs_v, [idx])
                o_v[pl.ds(i, L)] = x_v[pl.ds(i, L)] * g

        cz0.wait()
        cx0.wait()
        gather_scale(0, h)
        co0 = pltpu.make_async_copy(o_v.at[pl.ds(0, h)],
                                    out_hbm.at[pl.ds(base, h)], sem_o0)
        co0.start()

        cz1.wait()
        cx1.wait()
        gather_scale(h, n)
        co1 = pltpu.make_async_copy(o_v.at[pl.ds(h, n - h)],
                                    out_hbm.at[pl.ds(base + h, n - h)], sem_o1)
        co1.start()

        co0.wait()
        co1.wait()

    @pl.when(wid < NW - 1)
    def _():
        run(CHUNK)

    @pl.when(wid == NW - 1)
    def _():
        run(LAST)


@jax.jit
def kernel(x, atomic_numbers, scales):
    z = atomic_numbers.astype(jnp.int32)
    tab = jnp.reshape(scales, (NZ,))
    run = pl.kernel(
        _scale_body,
        mesh=plsc.VectorSubcoreMesh(core_axis_name="c", subcore_axis_name="s"),
        out_type=jax.ShapeDtypeStruct((N,), jnp.float32),
        compiler_params=pltpu.CompilerParams(needs_layout_passes=False),
        scratch_types=[
            pltpu.VMEM((CHUNK,), jnp.float32),
            pltpu.VMEM((CHUNK,), jnp.int32),
            pltpu.VMEM((TAB,), jnp.float32),
            pltpu.VMEM((TAB,), jnp.float32),
            pltpu.VMEM((CHUNK,), jnp.float32),
            pltpu.SemaphoreType.DMA,
            pltpu.SemaphoreType.DMA,
            pltpu.SemaphoreType.DMA,
            pltpu.SemaphoreType.DMA,
            pltpu.SemaphoreType.DMA,
            pltpu.SemaphoreType.DMA,
            pltpu.SemaphoreType.DMA,
        ],
    )
    return run(x, z, tab)


# speedup vs baseline: 1.0469x; 1.0469x over previous
import jax
import jax.numpy as jnp
from jax import lax
from jax.experimental import pallas as pl
from jax.experimental.pallas import tpu as pltpu
from jax.experimental.pallas import tpu_sc as plsc

N = 100000
NW = 32
CHUNK = 3136
LAST = N - (NW - 1) * CHUNK
NZ = 119
TAB = 128
L = 16


def _rsqrt16(s):
    y = jnp.where(s < jnp.float32(0.45), jnp.float32(1.8),
                  jnp.where(s < jnp.float32(1.8), jnp.float32(1.0),
                            jnp.float32(0.45)))
    for _ in range(6):
        y = y * (jnp.float32(1.5) - jnp.float32(0.5) * s * y * y)
    return y


def _scale_body(x_hbm, z_hbm, tab_hbm, out_hbm,
                x_v, z_v, tab_v, rs_v, o_v, sem_z, sem_x, sem_t):
    cid = lax.axis_index("c")
    sid = lax.axis_index("s")
    wid = sid * 2 + cid
    base = jnp.minimum(wid * CHUNK, N - CHUNK)

    cz = pltpu.make_async_copy(z_hbm.at[pl.ds(base, CHUNK)], z_v, sem_z)
    cx = pltpu.make_async_copy(x_hbm.at[pl.ds(base, CHUNK)], x_v, sem_x)
    ct = pltpu.make_async_copy(tab_hbm, tab_v.at[pl.ds(0, NZ)], sem_t)
    cz.start()
    cx.start()
    ct.start()

    ct.wait()
    for j in range(TAB // L):
        rs_v[pl.ds(j * L, L)] = _rsqrt16(tab_v[pl.ds(j * L, L)])

    cz.wait()
    cx.wait()

    @plsc.parallel_loop(0, CHUNK, step=L, unroll=8)
    def _body(i):
        idx = z_v[pl.ds(i, L)]
        g = plsc.load_gather(rs_v, [idx])
        o_v[pl.ds(i, L)] = x_v[pl.ds(i, L)] * g

    pltpu.sync_copy(o_v, out_hbm.at[pl.ds(base, CHUNK)])


@jax.jit
def kernel(x, atomic_numbers, scales):
    z = atomic_numbers.astype(jnp.int32)
    tab = jnp.reshape(scales, (NZ,))
    run = pl.kernel(
        _scale_body,
        mesh=plsc.VectorSubcoreMesh(core_axis_name="c", subcore_axis_name="s"),
        out_type=jax.ShapeDtypeStruct((N,), jnp.float32),
        compiler_params=pltpu.CompilerParams(needs_layout_passes=False),
        scratch_types=[
            pltpu.VMEM((CHUNK,), jnp.float32),
            pltpu.VMEM((CHUNK,), jnp.int32),
            pltpu.VMEM((TAB,), jnp.float32),
            pltpu.VMEM((TAB,), jnp.float32),
            pltpu.VMEM((CHUNK,), jnp.float32),
            pltpu.SemaphoreType.DMA,
            pltpu.SemaphoreType.DMA,
            pltpu.SemaphoreType.DMA,
        ],
    )
    return run(x, z, tab)
